# baseline (speedup 1.0000x reference)
import jax
import jax.numpy as jnp
from jax.experimental import pallas as pl
from jax.experimental.pallas import tpu as pltpu

_N = 1000
_NP = 1024
_NH = 512
_NC = 10
_NCP = 128
_K = _N + (_N * _N - _N - 1) // 2
_LAMBDA_SPEC = 0.05
_EPS = 1e-8
_TOTAL = _NP * _NP
_BIAS = -2147450880
_PAIR1 = 65537
_LO16 = 65535
_MAXF = 32767
_IMAX = 0x7FFFFFFF


def _cnt_ge_packed(xb, t):
    y = xb - t * _PAIR1
    z = jax.lax.shift_right_logical(y, 15) & _PAIR1
    colsum = jnp.sum(z, axis=0, keepdims=True)
    return jnp.sum((colsum & _LO16) + jax.lax.shift_right_logical(colsum, 16))


def _greedy15(count_lt, k_eff):
    p = jnp.int32(0)
    c = count_lt(jnp.int32(1 << 14))
    keep = c <= k_eff
    for r in range(1, 15):
        bit = jnp.int32(1 << (14 - r))
        bit_prev = jnp.int32(1 << (15 - r))
        c0 = count_lt(p | bit)
        c1 = count_lt(p | bit_prev | bit)
        p = jnp.where(keep, p | bit_prev, p)
        c = jnp.where(keep, c1, c0)
        keep = c <= k_eff
    return jnp.where(keep, p | 1, p)


def _spectral_loss_kernel(p_ref, u_ref, out_ref, col_s, row_s, u_s):
    col_s[:] = jnp.zeros((_NP, 8), jnp.float32)
    row_s[:] = jnp.zeros((8, _NP), jnp.float32)
    u_s[:] = jnp.zeros((_NP, _NCP), jnp.float32)
    col_s[0:_N, 0:3] = p_ref[:]
    row_s[0:3, 0:_N] = p_ref[:].T
    u_s[0:_N, 0:_NC] = u_ref[:]

    dx = col_s[:, 0:1] - row_s[0:1, :]
    dy = col_s[:, 1:2] - row_s[1:2, :]
    dz = col_s[:, 2:3] - row_s[2:3, :]
    sq = dx * dx + dy * dy + dz * dz

    rows = jax.lax.broadcasted_iota(jnp.int32, (_NP, _NP), 0)
    cols = jax.lax.broadcasted_iota(jnp.int32, (_NP, _NP), 1)
    valid = (rows < _N) & (cols < _N)

    bits = jax.lax.bitcast_convert_type(sq, jnp.int32)
    bits = jnp.where(valid, bits, _IMAX)

    a = bits[:, 0:_NH]
    b = bits[:, _NH:_NP]
    hi_a = jax.lax.shift_right_logical(a, 16)
    hi_b = jax.lax.shift_right_logical(b, 16)
    xh = (hi_a | jax.lax.shift_left(hi_b, 16)) + _BIAS

    def cnt_lt_hi(t):
        return _TOTAL - _cnt_ge_packed(xh, t)

    h = _greedy15(cnt_lt_hi, jnp.int32(_K))

    base = cnt_lt_hi(h)
    lo_a = jax.lax.shift_right_logical(a, 1) & _MAXF
    lo_b = jax.lax.shift_right_logical(b, 1) & _MAXF
    fa = jnp.where(hi_a == h, lo_a, _MAXF)
    fb = jnp.where(hi_b == h, lo_b, _MAXF)
    xl = (fa | jax.lax.shift_left(fb, 16)) + _BIAS

    def cnt_lt_lo(t):
        return _TOTAL - _cnt_ge_packed(xl, t)

    s = _greedy15(cnt_lt_lo, jnp.int32(_K) - base)

    resp = jax.lax.shift_left(h, 16) | jax.lax.shift_left(s, 1)
    trial = resp | 1
    cnt = -jnp.sum(jax.lax.shift_right_arithmetic(bits - trial, 31))
    res = jnp.where(cnt <= _K, trial, resp)

    sigma_sq = jax.lax.bitcast_convert_type(res, jnp.float32)
    sigma = jnp.sqrt(sigma_sq)
    denom = 2.0 * sigma * sigma + _EPS

    mask_w = valid & (rows != cols)
    w = jnp.where(mask_w, jnp.exp(-sq / denom), 0.0)

    d = jnp.sum(w, axis=1, keepdims=True)
    dinv = 1.0 / (jnp.sqrt(d) + _EPS)
    u = u_s[:] * dinv
    v = jnp.dot(w, u, preferred_element_type=jnp.float32)
    s2 = jnp.sum(v * u)
    s1 = jnp.sum(u_s[:] * u_s[:])
    total = (s1 - s2) / _NC
    loss = _LAMBDA_SPEC * total / (_N * _N)
    out_ref[0] = loss


@jax.jit
def kernel(points, outputs):
    out = pl.pallas_call(
        _spectral_loss_kernel,
        out_shape=jax.ShapeDtypeStruct((1,), jnp.float32),
        out_specs=pl.BlockSpec(memory_space=pltpu.SMEM),
        scratch_shapes=[
            pltpu.VMEM((_NP, 8), jnp.float32),
            pltpu.VMEM((8, _NP), jnp.float32),
            pltpu.VMEM((_NP, _NCP), jnp.float32),
        ],
    )(points, outputs)
    return out[0]
